# baseline (speedup 1.0000x reference)
import jax
import jax.numpy as jnp
from jax.experimental import pallas as pl

B = 4096
F = 128 * 128
NUM_ILLU = 1024
B_BLK = 128


def _scale_mul_kernel(idx_ref, w_ref, e_ref, out_ref):
    ids = jax.lax.broadcasted_iota(jnp.int32, (B_BLK, NUM_ILLU), 1)
    onehot = ids == idx_ref[...]
    scale = jnp.sum(jnp.where(onehot, w_ref[...], 0.0), axis=1, keepdims=True)
    out_ref[...] = e_ref[...] * scale


def kernel(E_in, idx, weights):
    e2 = E_in.reshape(B, F)
    idx2 = idx.reshape(B, 1).astype(jnp.int32)
    w2 = weights.reshape(1, NUM_ILLU)
    out = pl.pallas_call(
        _scale_mul_kernel,
        grid=(B // B_BLK,),
        in_specs=[
            pl.BlockSpec((B_BLK, 1), lambda i: (i, 0)),
            pl.BlockSpec((1, NUM_ILLU), lambda i: (0, 0)),
            pl.BlockSpec((B_BLK, F), lambda i: (i, 0)),
        ],
        out_specs=pl.BlockSpec((B_BLK, F), lambda i: (i, 0)),
        out_shape=jax.ShapeDtypeStruct((B, F), jnp.float32),
    )(idx2, w2, e2)
    return out.reshape(B, 128, 128)


# speedup vs baseline: 1.0031x; 1.0031x over previous
import jax
import jax.numpy as jnp
from jax import lax
from jax.experimental import pallas as pl
from jax.experimental.pallas import tpu as pltpu

B = 4096
F = 128 * 128
NUM_ILLU = 1024
B_BLK = 64
NBUF = 4
NSTEP = B // B_BLK


def _body(idx_ref, w_ref, e_hbm, o_hbm, ebuf, obuf, in_sems, out_sems):
    def in_copy(i, slot):
        return pltpu.make_async_copy(
            e_hbm.at[pl.ds(i * B_BLK, B_BLK)], ebuf.at[slot], in_sems.at[slot]
        )

    def out_copy(i, slot):
        return pltpu.make_async_copy(
            obuf.at[slot], o_hbm.at[pl.ds(i * B_BLK, B_BLK)], out_sems.at[slot]
        )

    for j in range(NBUF):
        in_copy(j, j).start()

    def step(i, carry):
        slot = lax.rem(i, NBUF)
        in_copy(i, slot).wait()

        @pl.when(i >= NBUF)
        def _():
            out_copy(i - NBUF, slot).wait()

        idx_blk = idx_ref[pl.ds(i * B_BLK, B_BLK), :]
        ids = lax.broadcasted_iota(jnp.int32, (B_BLK, NUM_ILLU), 1)
        scale = jnp.sum(
            jnp.where(ids == idx_blk, w_ref[...], 0.0), axis=1, keepdims=True
        )
        obuf[slot] = ebuf[slot] * scale
        out_copy(i, slot).start()

        @pl.when(i + NBUF < NSTEP)
        def _():
            in_copy(i + NBUF, slot).start()

        return carry

    lax.fori_loop(0, NSTEP, step, 0)

    for j in range(NBUF):
        out_copy(NSTEP - NBUF + j, j).wait()


def kernel(E_in, idx, weights):
    e2 = E_in.reshape(B, F)
    idx2 = idx.reshape(B, 1).astype(jnp.int32)
    w2 = weights.reshape(1, NUM_ILLU)
    out = pl.pallas_call(
        _body,
        in_specs=[
            pl.BlockSpec((B, 1), lambda: (0, 0)),
            pl.BlockSpec((1, NUM_ILLU), lambda: (0, 0)),
            pl.BlockSpec(memory_space=pl.ANY),
        ],
        out_specs=pl.BlockSpec(memory_space=pl.ANY),
        out_shape=jax.ShapeDtypeStruct((B, F), jnp.float32),
        scratch_shapes=[
            pltpu.VMEM((NBUF, B_BLK, F), jnp.float32),
            pltpu.VMEM((NBUF, B_BLK, F), jnp.float32),
            pltpu.SemaphoreType.DMA((NBUF,)),
            pltpu.SemaphoreType.DMA((NBUF,)),
        ],
    )(idx2, w2, e2)
    return out.reshape(B, 128, 128)


# speedup vs baseline: 3.2486x; 3.2387x over previous
import jax
import jax.numpy as jnp
from jax import lax
from jax.experimental import pallas as pl
from jax.experimental.pallas import tpu as pltpu

B = 4096
H = 128
NUM_ILLU = 1024
B_BLK = 64
NBUF = 4
NSTEP = B // B_BLK


def _body(idx_ref, w_ref, e_hbm, o_hbm, ebuf, obuf, in_sems, out_sems):
    def in_copy(i, slot):
        return pltpu.make_async_copy(
            e_hbm.at[pl.ds(i * B_BLK, B_BLK)], ebuf.at[slot], in_sems.at[slot]
        )

    def out_copy(i, slot):
        return pltpu.make_async_copy(
            obuf.at[slot], o_hbm.at[pl.ds(i * B_BLK, B_BLK)], out_sems.at[slot]
        )

    for j in range(NBUF):
        in_copy(j, j).start()

    def step(i, carry):
        slot = lax.rem(i, NBUF)
        in_copy(i, slot).wait()

        @pl.when(i >= NBUF)
        def _():
            out_copy(i - NBUF, slot).wait()

        idx_blk = idx_ref[pl.ds(i * B_BLK, B_BLK), :, :]
        ids = lax.broadcasted_iota(jnp.int32, (B_BLK, 1, NUM_ILLU), 2)
        scale = jnp.sum(
            jnp.where(ids == idx_blk, w_ref[...], 0.0), axis=2, keepdims=True
        )
        obuf[slot] = ebuf[slot] * scale
        out_copy(i, slot).start()

        @pl.when(i + NBUF < NSTEP)
        def _():
            in_copy(i + NBUF, slot).start()

        return carry

    lax.fori_loop(0, NSTEP, step, 0)

    for j in range(NBUF):
        out_copy(NSTEP - NBUF + j, j).wait()


def kernel(E_in, idx, weights):
    idx3 = idx.astype(jnp.int32)
    w3 = weights.reshape(1, 1, NUM_ILLU)
    out = pl.pallas_call(
        _body,
        in_specs=[
            pl.BlockSpec((B, 1, 1), lambda: (0, 0, 0)),
            pl.BlockSpec((1, 1, NUM_ILLU), lambda: (0, 0, 0)),
            pl.BlockSpec(memory_space=pl.ANY),
        ],
        out_specs=pl.BlockSpec(memory_space=pl.ANY),
        out_shape=jax.ShapeDtypeStruct((B, H, H), jnp.float32),
        scratch_shapes=[
            pltpu.VMEM((NBUF, B_BLK, H, H), jnp.float32),
            pltpu.VMEM((NBUF, B_BLK, H, H), jnp.float32),
            pltpu.SemaphoreType.DMA((NBUF,)),
            pltpu.SemaphoreType.DMA((NBUF,)),
        ],
    )(idx3, w3, E_in)
    return out
